# baseline (speedup 1.0000x reference)
import functools

import jax
import jax.numpy as jnp
from jax import lax
from jax.experimental import pallas as pl
from jax.experimental.pallas import tpu as pltpu
from jax.experimental.pallas import tpu_sc as plsc

GRID3 = 16
NUM_VOX = GRID3 ** 3
LANES = 16
PG = 2
CHUNK = 1024


def _sc_scatter(feat, ids_flat, B, DG, N):
    npg = N // PG
    nfull = npg // CHUNK
    tail = npg - nfull * CHUNK
    assert tail % LANES == 0
    W = (CHUNK + 127 + 127) // 128 * 128
    WT = (tail + 127 + 127) // 128 * 128 if tail else 0
    assert ((PG - 1) * npg + nfull * CHUNK) // 128 * 128 + WT \
        <= (N + 127) // 128 * 128
    mesh = plsc.VectorSubcoreMesh(core_axis_name="c", subcore_axis_name="s")

    @functools.partial(
        pl.kernel,
        out_type=[
            jax.ShapeDtypeStruct((B * DG * PG * LANES * NUM_VOX,), jnp.float32),
            jax.ShapeDtypeStruct((B * PG * NUM_VOX,), jnp.int32),
        ],
        mesh=mesh,
        compiler_params=pltpu.CompilerParams(needs_layout_passes=False),
        scratch_types=(
            [pltpu.VMEM((NUM_VOX,), jnp.float32)] * LANES
            + [
                pltpu.VMEM((LANES, W), jnp.float32),
                pltpu.VMEM((LANES, W), jnp.float32),
                pltpu.VMEM((CHUNK,), jnp.int32),
                pltpu.VMEM((CHUNK,), jnp.int32),
                pltpu.VMEM((NUM_VOX,), jnp.int32),
                pltpu.VMEM((NUM_VOX,), jnp.int32),
                pltpu.VMEM((LANES,), jnp.float32),
                pltpu.VMEM((LANES,), jnp.int32),
                pltpu.VMEM((CHUNK // LANES,), jnp.int32),
                pltpu.SMEM((1,), jnp.int32),
                pltpu.SemaphoreType.DMA,
                pltpu.SemaphoreType.DMA,
                pltpu.SemaphoreType.DMA,
                pltpu.SemaphoreType.DMA,
            ]
        ),
    )
    def sc_kernel(feat_ref, ids_ref, outmax_ref, outcnt_ref, *scr):
        accs = scr[:LANES]
        (fbuf0, fbuf1, idbuf0, idbuf1, cnt, marks, tmpf, tmpk,
         wl, wlcnt, fsem0, fsem1, isem0, isem1) = scr[LANES:]
        fbufs, idbufs = (fbuf0, fbuf1), (idbuf0, idbuf1)
        fsems, isems = (fsem0, fsem1), (isem0, isem1)

        wid = lax.axis_index("s") * 2 + lax.axis_index("c")
        b = wid // (DG * PG)
        dg = (wid // PG) % DG
        pg = wid % PG
        do_cnt = dg == 0

        iota16 = lax.iota(jnp.int32, LANES)
        neg_inf = jnp.full((LANES,), -jnp.inf, jnp.float32)
        zeros16 = jnp.zeros((LANES,), jnp.int32)
        ones16 = jnp.ones((LANES,), jnp.int32)

        def init_acc(v, _):
            off = pl.ds(pl.multiple_of(v * LANES, LANES), LANES)
            for d in range(LANES):
                accs[d][off] = neg_inf
            return 0
        lax.fori_loop(0, NUM_VOX // LANES, init_acc, 0)

        @pl.when(do_cnt)
        def _():
            def init_cnt(k, _):
                cnt[pl.ds(pl.multiple_of(k * LANES, LANES), LANES)] = zeros16
                return 0
            lax.fori_loop(0, NUM_VOX // LANES, init_cnt, 0)

        pbase = pg * npg
        dgbase = pl.multiple_of(dg * LANES, LANES)
        sh_idx = [jnp.maximum(iota16 - s, 0) for s in (1, 2, 4, 8)]
        nxt_idx = jnp.minimum(iota16 + 1, LANES - 1)

        def chunk_refs(c, par, w, npts):
            pt0 = pl.multiple_of(pbase + c * CHUNK, 16)
            pt0a = pl.multiple_of((pt0 // 128) * 128, 128)
            i_off = pl.multiple_of(b * N + pt0, 16)
            fsrc = feat_ref.at[b, pl.ds(dgbase, LANES), pl.ds(pt0a, w)]
            fdst = fbufs[par] if w == W else fbufs[par].at[:, pl.ds(0, w)]
            isrc = ids_ref.at[pl.ds(i_off, npts)]
            idst = (idbufs[par] if npts == CHUNK
                    else idbufs[par].at[pl.ds(0, npts)])
            return fsrc, fdst, isrc, idst

        def start_chunk(c, par, w=W, npts=CHUNK):
            fsrc, fdst, isrc, idst = chunk_refs(c, par, w, npts)
            pltpu.async_copy(fsrc, fdst, fsems[par])
            pltpu.async_copy(isrc, idst, isems[par])

        def process_chunk(c, par, w=W, npts=CHUNK):
            fsrc, fdst, isrc, idst = chunk_refs(c, par, w, npts)
            pltpu.make_async_copy(fsrc, fdst, fsems[par]).wait()
            pltpu.make_async_copy(isrc, idst, isems[par]).wait()
            fbuf, idbuf = fbufs[par], idbufs[par]
            pt0 = pl.multiple_of(pbase + c * CHUNK, 16)
            pt0a = pl.multiple_of((pt0 // 128) * 128, 128)
            shift = pt0 - pt0a

            @pl.when(do_cnt)
            def _():
                def cnt_body(k, _):
                    ids16 = idbuf[pl.ds(pl.multiple_of(k * LANES, LANES),
                                        LANES)]
                    plsc.addupdate_scatter(cnt, [ids16], ones16)
                    return 0
                lax.fori_loop(0, npts // LANES, cnt_body, 0)

            wlcnt[0] = 0

            def group_body(k, _):
                i0 = pl.multiple_of(k * LANES, LANES)
                ids16 = idbuf[pl.ds(i0, LANES)]
                col0 = pl.multiple_of(shift + i0, 16)

                plsc.store_scatter(marks, [ids16], iota16)
                got = plsc.load_gather(marks, [ids16])
                dup = plsc.all_reduce_population_count(got != iota16)
                nw = wlcnt[0]
                plsc.store_scatter(wl, [jnp.full((LANES,), nw, jnp.int32)],
                                   jnp.full((LANES,), k, jnp.int32),
                                   mask=(dup > 0) & (iota16 == 0))
                wlcnt[0] = nw + jnp.minimum(dup[0], 1)

                for d in range(LANES):
                    f16 = fbuf[d, pl.ds(col0, LANES)]
                    a = plsc.load_gather(accs[d], [ids16])
                    plsc.store_scatter(accs[d], [ids16],
                                       jnp.maximum(a, f16))
                return 0

            lax.fori_loop(0, npts // LANES, group_body, 0)

            def fix_body(w, _):
                k = plsc.load_gather(wl, [jnp.full((LANES,), w, jnp.int32)])[0]
                i0 = pl.multiple_of(k * LANES, LANES)
                ids16 = idbuf[pl.ds(i0, LANES)]
                col0 = pl.multiple_of(shift + i0, 16)
                skeys, sperm = plsc.sort_key_val(ids16, iota16)
                tmpk[...] = skeys
                eqm = []
                for s, idx in zip((1, 2, 4, 8), sh_idx):
                    shk = plsc.load_gather(tmpk, [idx])
                    eqm.append((skeys == shk) & (iota16 >= s))
                nxtk = plsc.load_gather(tmpk, [nxt_idx])
                last = (skeys != nxtk) | (iota16 == LANES - 1)
                for d in range(LANES):
                    f16 = fbuf[d, pl.ds(col0, LANES)]
                    tmpf[...] = f16
                    pf = plsc.load_gather(tmpf, [sperm])
                    for eq, idx in zip(eqm, sh_idx):
                        tmpf[...] = pf
                        shv = plsc.load_gather(tmpf, [idx])
                        pf = jnp.where(eq, jnp.maximum(pf, shv), pf)
                    a = plsc.load_gather(accs[d], [skeys])
                    plsc.store_scatter(accs[d], [skeys],
                                       jnp.maximum(a, pf), mask=last)
                return 0

            lax.fori_loop(0, wlcnt[0], fix_body, 0)

        start_chunk(0, 0)
        start_chunk(1, 1)

        def pair_body(g, _):
            for par in range(2):
                c = g * 2 + par
                process_chunk(c, par)

                @pl.when(c + 2 < nfull)
                def _():
                    start_chunk(c + 2, par)
                if tail:
                    @pl.when(c + 2 == nfull)
                    def _():
                        start_chunk(nfull, par, w=WT, npts=tail)
            return 0

        lax.fori_loop(0, nfull // 2, pair_body, 0)
        if nfull % 2:
            process_chunk(nfull - 1, (nfull - 1) % 2)
        if tail:
            process_chunk(nfull, nfull % 2, w=WT, npts=tail)

        obase = ((b * DG + dg) * PG + pg) * LANES * NUM_VOX
        for d in range(LANES):
            o_off = pl.multiple_of(obase + d * NUM_VOX, 128)
            pltpu.sync_copy(accs[d], outmax_ref.at[pl.ds(o_off, NUM_VOX)])

        @pl.when(do_cnt)
        def _():
            c_off = pl.multiple_of((b * PG + pg) * NUM_VOX, 128)
            pltpu.sync_copy(cnt, outcnt_ref.at[pl.ds(c_off, NUM_VOX)])

    return sc_kernel(feat, ids_flat)


def _tc_merge(partmax, partcnt, B, DG):
    def body(pm_ref, pc_ref, vf_ref, cnt_ref):
        m = jnp.maximum(pm_ref[0, 0, 0], pm_ref[0, 0, 1])
        c = pc_ref[0, 0] + pc_ref[0, 1]
        vf_ref[0, 0] = jnp.where((c > 0)[None, :], m, 0.0)
        cnt_ref[0, 0] = c

    return pl.pallas_call(
        body,
        grid=(B, DG),
        in_specs=[
            pl.BlockSpec((1, 1, PG, LANES, NUM_VOX),
                         lambda b, d: (b, d, 0, 0, 0)),
            pl.BlockSpec((1, PG, NUM_VOX), lambda b, d: (b, 0, 0)),
        ],
        out_specs=[
            pl.BlockSpec((1, 1, LANES, NUM_VOX), lambda b, d: (b, d, 0, 0)),
            pl.BlockSpec((1, 1, NUM_VOX), lambda b, d: (b, 0, 0)),
        ],
        out_shape=[
            jax.ShapeDtypeStruct((B, DG, LANES, NUM_VOX), jnp.float32),
            jax.ShapeDtypeStruct((B, 1, NUM_VOX), jnp.int32),
        ],
    )(partmax, partcnt)


def kernel(features, voxel_ids):
    B, D, N = features.shape
    DG = D // LANES

    partmax_flat, partcnt_flat = _sc_scatter(
        features, voxel_ids.reshape(-1), B, DG, N)
    partmax = partmax_flat.reshape(B, DG, PG, LANES, NUM_VOX)
    partcnt = partcnt_flat.reshape(B, PG, NUM_VOX)
    vf, counts = _tc_merge(partmax, partcnt, B, DG)

    vf3d = vf.reshape(B, D, GRID3, GRID3, GRID3)
    return vf3d, voxel_ids, counts


# speedup vs baseline: 1.0528x; 1.0528x over previous
import functools

import jax
import jax.numpy as jnp
from jax import lax
from jax.experimental import pallas as pl
from jax.experimental.pallas import tpu as pltpu
from jax.experimental.pallas import tpu_sc as plsc

GRID3 = 16
NUM_VOX = GRID3 ** 3
LANES = 16
PG = 2
CHUNK = 1024


def _sc_scatter(feat, ids_flat, B, DG, N):
    npg = N // PG
    nfull = npg // CHUNK
    tail = npg - nfull * CHUNK
    assert tail % LANES == 0
    W = (CHUNK + 127 + 127) // 128 * 128
    WT = (tail + 127 + 127) // 128 * 128 if tail else 0
    assert ((PG - 1) * npg + nfull * CHUNK) // 128 * 128 + WT \
        <= (N + 127) // 128 * 128
    mesh = plsc.VectorSubcoreMesh(core_axis_name="c", subcore_axis_name="s")

    @functools.partial(
        pl.kernel,
        out_type=[
            jax.ShapeDtypeStruct((B * DG * PG * LANES * NUM_VOX,), jnp.float32),
            jax.ShapeDtypeStruct((B * PG * NUM_VOX,), jnp.int32),
        ],
        mesh=mesh,
        compiler_params=pltpu.CompilerParams(needs_layout_passes=False),
        scratch_types=(
            [pltpu.VMEM((NUM_VOX,), jnp.float32)] * LANES
            + [
                pltpu.VMEM((LANES, W), jnp.float32),
                pltpu.VMEM((LANES, W), jnp.float32),
                pltpu.VMEM((CHUNK,), jnp.int32),
                pltpu.VMEM((CHUNK,), jnp.int32),
                pltpu.VMEM((NUM_VOX,), jnp.int32),
                pltpu.VMEM((NUM_VOX,), jnp.int32),
                pltpu.VMEM((LANES,), jnp.float32),
                pltpu.VMEM((LANES,), jnp.int32),
                pltpu.SemaphoreType.DMA,
                pltpu.SemaphoreType.DMA,
                pltpu.SemaphoreType.DMA,
                pltpu.SemaphoreType.DMA,
            ]
        ),
    )
    def sc_kernel(feat_ref, ids_ref, outmax_ref, outcnt_ref, *scr):
        accs = scr[:LANES]
        (fbuf0, fbuf1, idbuf0, idbuf1, cnt, marks, tmpf, tmpk,
         fsem0, fsem1, isem0, isem1) = scr[LANES:]
        fbufs, idbufs = (fbuf0, fbuf1), (idbuf0, idbuf1)
        fsems, isems = (fsem0, fsem1), (isem0, isem1)

        wid = lax.axis_index("s") * 2 + lax.axis_index("c")
        b = wid // (DG * PG)
        dg = (wid // PG) % DG
        pg = wid % PG
        do_cnt = dg == 0

        iota16 = lax.iota(jnp.int32, LANES)
        neg_inf = jnp.full((LANES,), -jnp.inf, jnp.float32)
        zeros16 = jnp.zeros((LANES,), jnp.int32)
        ones16 = jnp.ones((LANES,), jnp.int32)

        def init_acc(v, _):
            off = pl.ds(pl.multiple_of(v * LANES, LANES), LANES)
            for d in range(LANES):
                accs[d][off] = neg_inf
            return 0
        lax.fori_loop(0, NUM_VOX // LANES, init_acc, 0)

        @pl.when(do_cnt)
        def _():
            def init_cnt(k, _):
                cnt[pl.ds(pl.multiple_of(k * LANES, LANES), LANES)] = zeros16
                return 0
            lax.fori_loop(0, NUM_VOX // LANES, init_cnt, 0)

        pbase = pg * npg
        dgbase = pl.multiple_of(dg * LANES, LANES)
        sh_idx = [jnp.maximum(iota16 - s, 0) for s in (1, 2, 4, 8)]
        nxt_idx = jnp.minimum(iota16 + 1, LANES - 1)

        def chunk_refs(c, par, w, npts):
            pt0 = pl.multiple_of(pbase + c * CHUNK, 16)
            pt0a = pl.multiple_of((pt0 // 128) * 128, 128)
            i_off = pl.multiple_of(b * N + pt0, 16)
            fsrc = feat_ref.at[b, pl.ds(dgbase, LANES), pl.ds(pt0a, w)]
            fdst = fbufs[par] if w == W else fbufs[par].at[:, pl.ds(0, w)]
            isrc = ids_ref.at[pl.ds(i_off, npts)]
            idst = (idbufs[par] if npts == CHUNK
                    else idbufs[par].at[pl.ds(0, npts)])
            return fsrc, fdst, isrc, idst

        def start_chunk(c, par, w=W, npts=CHUNK):
            fsrc, fdst, isrc, idst = chunk_refs(c, par, w, npts)
            pltpu.async_copy(fsrc, fdst, fsems[par])
            pltpu.async_copy(isrc, idst, isems[par])

        def process_chunk(c, par, w=W, npts=CHUNK):
            fsrc, fdst, isrc, idst = chunk_refs(c, par, w, npts)
            pltpu.make_async_copy(fsrc, fdst, fsems[par]).wait()
            pltpu.make_async_copy(isrc, idst, isems[par]).wait()
            fbuf, idbuf = fbufs[par], idbufs[par]
            pt0 = pl.multiple_of(pbase + c * CHUNK, 16)
            pt0a = pl.multiple_of((pt0 // 128) * 128, 128)
            shift = pt0 - pt0a

            @pl.when(do_cnt)
            def _():
                def cnt_body(k, _):
                    ids16 = idbuf[pl.ds(pl.multiple_of(k * LANES, LANES),
                                        LANES)]
                    plsc.addupdate_scatter(cnt, [ids16], ones16)
                    return 0
                lax.fori_loop(0, npts // LANES, cnt_body, 0)

            def group_body(k, _):
                i0 = pl.multiple_of(k * LANES, LANES)
                ids16 = idbuf[pl.ds(i0, LANES)]
                col0 = pl.multiple_of(shift + i0, 16)

                plsc.store_scatter(marks, [ids16], iota16)
                got = plsc.load_gather(marks, [ids16])
                nuniq = plsc.all_reduce_population_count(got == iota16)[0]

                for d in range(LANES):
                    f16 = fbuf[d, pl.ds(col0, LANES)]
                    a = plsc.load_gather(accs[d], [ids16])
                    plsc.store_scatter(accs[d], [ids16],
                                       jnp.maximum(a, f16))

                @pl.when(nuniq < LANES)
                def _():
                    skeys, sperm = plsc.sort_key_val(ids16, iota16)
                    tmpk[...] = skeys
                    eqm = []
                    for s, idx in zip((1, 2, 4, 8), sh_idx):
                        shk = plsc.load_gather(tmpk, [idx])
                        eqm.append((skeys == shk) & (iota16 >= s))
                    nxtk = plsc.load_gather(tmpk, [nxt_idx])
                    last = (skeys != nxtk) | (iota16 == LANES - 1)
                    for d in range(LANES):
                        f16 = fbuf[d, pl.ds(col0, LANES)]
                        tmpf[...] = f16
                        pf = plsc.load_gather(tmpf, [sperm])
                        for eq, idx in zip(eqm, sh_idx):
                            tmpf[...] = pf
                            shv = plsc.load_gather(tmpf, [idx])
                            pf = jnp.where(eq, jnp.maximum(pf, shv), pf)
                        a = plsc.load_gather(accs[d], [skeys])
                        plsc.store_scatter(accs[d], [skeys],
                                           jnp.maximum(a, pf), mask=last)
                return 0

            lax.fori_loop(0, npts // LANES, group_body, 0)

        start_chunk(0, 0)
        start_chunk(1, 1)

        def pair_body(g, _):
            for par in range(2):
                c = g * 2 + par
                process_chunk(c, par)

                @pl.when(c + 2 < nfull)
                def _():
                    start_chunk(c + 2, par)
                if tail:
                    @pl.when(c + 2 == nfull)
                    def _():
                        start_chunk(nfull, par, w=WT, npts=tail)
            return 0

        lax.fori_loop(0, nfull // 2, pair_body, 0)
        if nfull % 2:
            process_chunk(nfull - 1, (nfull - 1) % 2)
        if tail:
            process_chunk(nfull, nfull % 2, w=WT, npts=tail)

        obase = ((b * DG + dg) * PG + pg) * LANES * NUM_VOX
        for d in range(LANES):
            o_off = pl.multiple_of(obase + d * NUM_VOX, 128)
            pltpu.sync_copy(accs[d], outmax_ref.at[pl.ds(o_off, NUM_VOX)])

        @pl.when(do_cnt)
        def _():
            c_off = pl.multiple_of((b * PG + pg) * NUM_VOX, 128)
            pltpu.sync_copy(cnt, outcnt_ref.at[pl.ds(c_off, NUM_VOX)])

    return sc_kernel(feat, ids_flat)


def _tc_merge(partmax, partcnt, B, DG):
    def body(pm_ref, pc_ref, vf_ref, cnt_ref):
        m = jnp.maximum(pm_ref[0, 0, 0], pm_ref[0, 0, 1])
        c = pc_ref[0, 0] + pc_ref[0, 1]
        vf_ref[0, 0] = jnp.where((c > 0)[None, :], m, 0.0)
        cnt_ref[0, 0] = c

    return pl.pallas_call(
        body,
        grid=(B, DG),
        in_specs=[
            pl.BlockSpec((1, 1, PG, LANES, NUM_VOX),
                         lambda b, d: (b, d, 0, 0, 0)),
            pl.BlockSpec((1, PG, NUM_VOX), lambda b, d: (b, 0, 0)),
        ],
        out_specs=[
            pl.BlockSpec((1, 1, LANES, NUM_VOX), lambda b, d: (b, d, 0, 0)),
            pl.BlockSpec((1, 1, NUM_VOX), lambda b, d: (b, 0, 0)),
        ],
        out_shape=[
            jax.ShapeDtypeStruct((B, DG, LANES, NUM_VOX), jnp.float32),
            jax.ShapeDtypeStruct((B, 1, NUM_VOX), jnp.int32),
        ],
    )(partmax, partcnt)


def kernel(features, voxel_ids):
    B, D, N = features.shape
    DG = D // LANES

    partmax_flat, partcnt_flat = _sc_scatter(
        features, voxel_ids.reshape(-1), B, DG, N)
    partmax = partmax_flat.reshape(B, DG, PG, LANES, NUM_VOX)
    partcnt = partcnt_flat.reshape(B, PG, NUM_VOX)
    vf, counts = _tc_merge(partmax, partcnt, B, DG)

    vf3d = vf.reshape(B, D, GRID3, GRID3, GRID3)
    return vf3d, voxel_ids, counts


# speedup vs baseline: 1.0940x; 1.0391x over previous
import functools

import jax
import jax.numpy as jnp
from jax import lax
from jax.experimental import pallas as pl
from jax.experimental.pallas import tpu as pltpu
from jax.experimental.pallas import tpu_sc as plsc

GRID3 = 16
NUM_VOX = GRID3 ** 3
LANES = 16
PG = 2
CHUNK = 1024


def _sc_scatter(feat, ids_flat, B, DG, N):
    npg = N // PG
    nfull = npg // CHUNK
    tail = npg - nfull * CHUNK
    assert tail % LANES == 0
    W = (CHUNK + 127 + 127) // 128 * 128
    WT = (tail + 127 + 127) // 128 * 128 if tail else 0
    assert ((PG - 1) * npg + nfull * CHUNK) // 128 * 128 + WT \
        <= (N + 127) // 128 * 128
    mesh = plsc.VectorSubcoreMesh(core_axis_name="c", subcore_axis_name="s")

    @functools.partial(
        pl.kernel,
        out_type=[
            jax.ShapeDtypeStruct((B * DG * PG * LANES * NUM_VOX,), jnp.float32),
            jax.ShapeDtypeStruct((B * PG * NUM_VOX,), jnp.int32),
        ],
        mesh=mesh,
        compiler_params=pltpu.CompilerParams(needs_layout_passes=False),
        scratch_types=(
            [pltpu.VMEM((NUM_VOX,), jnp.float32)] * LANES
            + [
                pltpu.VMEM((LANES, W), jnp.float32),
                pltpu.VMEM((LANES, W), jnp.float32),
                pltpu.VMEM((CHUNK,), jnp.int32),
                pltpu.VMEM((CHUNK,), jnp.int32),
                pltpu.VMEM((NUM_VOX,), jnp.int32),
                pltpu.VMEM((NUM_VOX,), jnp.int32),
                pltpu.VMEM((LANES,), jnp.float32),
                pltpu.VMEM((LANES,), jnp.int32),
                pltpu.SemaphoreType.DMA,
                pltpu.SemaphoreType.DMA,
                pltpu.SemaphoreType.DMA,
                pltpu.SemaphoreType.DMA,
            ]
        ),
    )
    def sc_kernel(feat_ref, ids_ref, outmax_ref, outcnt_ref, *scr):
        accs = scr[:LANES]
        (fbuf0, fbuf1, idbuf0, idbuf1, cnt, marks, tmpf, tmpk,
         fsem0, fsem1, isem0, isem1) = scr[LANES:]
        fbufs, idbufs = (fbuf0, fbuf1), (idbuf0, idbuf1)
        fsems, isems = (fsem0, fsem1), (isem0, isem1)

        wid = lax.axis_index("s") * 2 + lax.axis_index("c")
        b = wid // (DG * PG)
        dg = (wid // PG) % DG
        pg = wid % PG
        do_cnt = dg == 0

        iota16 = lax.iota(jnp.int32, LANES)
        neg_inf = jnp.full((LANES,), -jnp.inf, jnp.float32)
        zeros16 = jnp.zeros((LANES,), jnp.int32)
        ones16 = jnp.ones((LANES,), jnp.int32)

        def init_acc(v, _):
            off = pl.ds(pl.multiple_of(v * LANES, LANES), LANES)
            for d in range(LANES):
                accs[d][off] = neg_inf
            return 0
        lax.fori_loop(0, NUM_VOX // LANES, init_acc, 0)

        @pl.when(do_cnt)
        def _():
            def init_cnt(k, _):
                cnt[pl.ds(pl.multiple_of(k * LANES, LANES), LANES)] = zeros16
                return 0
            lax.fori_loop(0, NUM_VOX // LANES, init_cnt, 0)

        pbase = pg * npg
        dgbase = pl.multiple_of(dg * LANES, LANES)
        sh_idx = [jnp.maximum(iota16 - s, 0) for s in (1, 2, 4, 8)]
        nxt_idx = jnp.minimum(iota16 + 1, LANES - 1)

        def chunk_refs(c, par, w, npts):
            pt0 = pl.multiple_of(pbase + c * CHUNK, 16)
            pt0a = pl.multiple_of((pt0 // 128) * 128, 128)
            i_off = pl.multiple_of(b * N + pt0, 16)
            fsrc = feat_ref.at[b, pl.ds(dgbase, LANES), pl.ds(pt0a, w)]
            fdst = fbufs[par] if w == W else fbufs[par].at[:, pl.ds(0, w)]
            isrc = ids_ref.at[pl.ds(i_off, npts)]
            idst = (idbufs[par] if npts == CHUNK
                    else idbufs[par].at[pl.ds(0, npts)])
            return fsrc, fdst, isrc, idst

        def start_chunk(c, par, w=W, npts=CHUNK):
            fsrc, fdst, isrc, idst = chunk_refs(c, par, w, npts)
            pltpu.async_copy(fsrc, fdst, fsems[par])
            pltpu.async_copy(isrc, idst, isems[par])

        def process_chunk(c, par, w=W, npts=CHUNK):
            fsrc, fdst, isrc, idst = chunk_refs(c, par, w, npts)
            pltpu.make_async_copy(fsrc, fdst, fsems[par]).wait()
            pltpu.make_async_copy(isrc, idst, isems[par]).wait()
            fbuf, idbuf = fbufs[par], idbufs[par]
            pt0 = pl.multiple_of(pbase + c * CHUNK, 16)
            pt0a = pl.multiple_of((pt0 // 128) * 128, 128)
            shift = pt0 - pt0a

            def group_body(k, _):
                i0 = pl.multiple_of(k * LANES, LANES)
                ids16 = idbuf[pl.ds(i0, LANES)]
                col0 = pl.multiple_of(shift + i0, 16)

                @pl.when(do_cnt)
                def _():
                    plsc.addupdate_scatter(cnt, [ids16], ones16)

                plsc.store_scatter(marks, [ids16], iota16)
                got = plsc.load_gather(marks, [ids16])
                nuniq = plsc.all_reduce_population_count(got == iota16)[0]

                for d in range(LANES):
                    f16 = fbuf[d, pl.ds(col0, LANES)]
                    a = plsc.load_gather(accs[d], [ids16])
                    plsc.store_scatter(accs[d], [ids16],
                                       jnp.maximum(a, f16))

                @pl.when(nuniq < LANES)
                def _():
                    skeys, sperm = plsc.sort_key_val(ids16, iota16)
                    tmpk[...] = skeys
                    eqm = []
                    for s, idx in zip((1, 2, 4, 8), sh_idx):
                        shk = plsc.load_gather(tmpk, [idx])
                        eqm.append((skeys == shk) & (iota16 >= s))
                    nxtk = plsc.load_gather(tmpk, [nxt_idx])
                    last = (skeys != nxtk) | (iota16 == LANES - 1)
                    for d in range(LANES):
                        f16 = fbuf[d, pl.ds(col0, LANES)]
                        tmpf[...] = f16
                        pf = plsc.load_gather(tmpf, [sperm])
                        for eq, idx in zip(eqm, sh_idx):
                            tmpf[...] = pf
                            shv = plsc.load_gather(tmpf, [idx])
                            pf = jnp.where(eq, jnp.maximum(pf, shv), pf)
                        a = plsc.load_gather(accs[d], [skeys])
                        plsc.store_scatter(accs[d], [skeys],
                                           jnp.maximum(a, pf), mask=last)
                return 0

            lax.fori_loop(0, npts // LANES, group_body, 0, unroll=2)

        start_chunk(0, 0)
        start_chunk(1, 1)

        def pair_body(g, _):
            for par in range(2):
                c = g * 2 + par
                process_chunk(c, par)

                @pl.when(c + 2 < nfull)
                def _():
                    start_chunk(c + 2, par)
                if tail:
                    @pl.when(c + 2 == nfull)
                    def _():
                        start_chunk(nfull, par, w=WT, npts=tail)
            return 0

        lax.fori_loop(0, nfull // 2, pair_body, 0)
        if nfull % 2:
            process_chunk(nfull - 1, (nfull - 1) % 2)
        if tail:
            process_chunk(nfull, nfull % 2, w=WT, npts=tail)

        obase = ((b * DG + dg) * PG + pg) * LANES * NUM_VOX
        for d in range(LANES):
            o_off = pl.multiple_of(obase + d * NUM_VOX, 128)
            pltpu.sync_copy(accs[d], outmax_ref.at[pl.ds(o_off, NUM_VOX)])

        @pl.when(do_cnt)
        def _():
            c_off = pl.multiple_of((b * PG + pg) * NUM_VOX, 128)
            pltpu.sync_copy(cnt, outcnt_ref.at[pl.ds(c_off, NUM_VOX)])

    return sc_kernel(feat, ids_flat)


def _tc_merge(partmax, partcnt, B, DG):
    def body(pm_ref, pc_ref, vf_ref, cnt_ref):
        m = jnp.maximum(pm_ref[0, 0, 0], pm_ref[0, 0, 1])
        c = pc_ref[0, 0] + pc_ref[0, 1]
        vf_ref[0, 0] = jnp.where((c > 0)[None, :], m, 0.0)
        cnt_ref[0, 0] = c

    return pl.pallas_call(
        body,
        grid=(B, DG),
        in_specs=[
            pl.BlockSpec((1, 1, PG, LANES, NUM_VOX),
                         lambda b, d: (b, d, 0, 0, 0)),
            pl.BlockSpec((1, PG, NUM_VOX), lambda b, d: (b, 0, 0)),
        ],
        out_specs=[
            pl.BlockSpec((1, 1, LANES, NUM_VOX), lambda b, d: (b, d, 0, 0)),
            pl.BlockSpec((1, 1, NUM_VOX), lambda b, d: (b, 0, 0)),
        ],
        out_shape=[
            jax.ShapeDtypeStruct((B, DG, LANES, NUM_VOX), jnp.float32),
            jax.ShapeDtypeStruct((B, 1, NUM_VOX), jnp.int32),
        ],
    )(partmax, partcnt)


def kernel(features, voxel_ids):
    B, D, N = features.shape
    DG = D // LANES

    partmax_flat, partcnt_flat = _sc_scatter(
        features, voxel_ids.reshape(-1), B, DG, N)
    partmax = partmax_flat.reshape(B, DG, PG, LANES, NUM_VOX)
    partcnt = partcnt_flat.reshape(B, PG, NUM_VOX)
    vf, counts = _tc_merge(partmax, partcnt, B, DG)

    vf3d = vf.reshape(B, D, GRID3, GRID3, GRID3)
    return vf3d, voxel_ids, counts


# speedup vs baseline: 1.1043x; 1.0094x over previous
import functools

import jax
import jax.numpy as jnp
from jax import lax
from jax.experimental import pallas as pl
from jax.experimental.pallas import tpu as pltpu
from jax.experimental.pallas import tpu_sc as plsc

GRID3 = 16
NUM_VOX = GRID3 ** 3
LANES = 16
PG = 2
CHUNK = 1024


def _sc_scatter(feat, ids_flat, B, DG, N):
    npg = N // PG
    nfull = npg // CHUNK
    tail = npg - nfull * CHUNK
    assert tail % LANES == 0
    W = (CHUNK + 127 + 127) // 128 * 128
    WT = (tail + 127 + 127) // 128 * 128 if tail else 0
    assert ((PG - 1) * npg + nfull * CHUNK) // 128 * 128 + WT \
        <= (N + 127) // 128 * 128
    mesh = plsc.VectorSubcoreMesh(core_axis_name="c", subcore_axis_name="s")

    @functools.partial(
        pl.kernel,
        out_type=[
            jax.ShapeDtypeStruct((B * DG * PG * LANES * NUM_VOX,), jnp.float32),
            jax.ShapeDtypeStruct((B * PG * NUM_VOX,), jnp.int32),
        ],
        mesh=mesh,
        compiler_params=pltpu.CompilerParams(needs_layout_passes=False),
        scratch_types=(
            [pltpu.VMEM((NUM_VOX,), jnp.float32)] * LANES
            + [
                pltpu.VMEM((LANES, W), jnp.float32),
                pltpu.VMEM((LANES, W), jnp.float32),
                pltpu.VMEM((CHUNK,), jnp.int32),
                pltpu.VMEM((CHUNK,), jnp.int32),
                pltpu.VMEM((NUM_VOX,), jnp.int32),
                pltpu.VMEM((NUM_VOX,), jnp.int32),
                pltpu.VMEM((LANES,), jnp.float32),
                pltpu.VMEM((LANES,), jnp.int32),
                pltpu.SemaphoreType.DMA,
                pltpu.SemaphoreType.DMA,
                pltpu.SemaphoreType.DMA,
                pltpu.SemaphoreType.DMA,
            ]
        ),
    )
    def sc_kernel(feat_ref, ids_ref, outmax_ref, outcnt_ref, *scr):
        accs = scr[:LANES]
        (fbuf0, fbuf1, idbuf0, idbuf1, cnt, marks, tmpf, tmpk,
         fsem0, fsem1, isem0, isem1) = scr[LANES:]
        fbufs, idbufs = (fbuf0, fbuf1), (idbuf0, idbuf1)
        fsems, isems = (fsem0, fsem1), (isem0, isem1)

        wid = lax.axis_index("s") * 2 + lax.axis_index("c")
        b = wid // (DG * PG)
        dg = (wid // PG) % DG
        pg = wid % PG
        do_cnt = dg == 0

        iota16 = lax.iota(jnp.int32, LANES)
        neg_inf = jnp.full((LANES,), -jnp.inf, jnp.float32)
        zeros16 = jnp.zeros((LANES,), jnp.int32)
        ones16 = jnp.ones((LANES,), jnp.int32)

        def init_acc(v, _):
            off = pl.ds(pl.multiple_of(v * LANES, LANES), LANES)
            for d in range(LANES):
                accs[d][off] = neg_inf
            return 0
        lax.fori_loop(0, NUM_VOX // LANES, init_acc, 0)

        @pl.when(do_cnt)
        def _():
            def init_cnt(k, _):
                cnt[pl.ds(pl.multiple_of(k * LANES, LANES), LANES)] = zeros16
                return 0
            lax.fori_loop(0, NUM_VOX // LANES, init_cnt, 0)

        pbase = pg * npg
        dgbase = pl.multiple_of(dg * LANES, LANES)
        sh_idx = [jnp.maximum(iota16 - s, 0) for s in (1, 2, 4, 8)]
        nxt_idx = jnp.minimum(iota16 + 1, LANES - 1)

        def chunk_refs(c, par, w, npts):
            pt0 = pl.multiple_of(pbase + c * CHUNK, 16)
            pt0a = pl.multiple_of((pt0 // 128) * 128, 128)
            i_off = pl.multiple_of(b * N + pt0, 16)
            fsrc = feat_ref.at[b, pl.ds(dgbase, LANES), pl.ds(pt0a, w)]
            fdst = fbufs[par] if w == W else fbufs[par].at[:, pl.ds(0, w)]
            isrc = ids_ref.at[pl.ds(i_off, npts)]
            idst = (idbufs[par] if npts == CHUNK
                    else idbufs[par].at[pl.ds(0, npts)])
            return fsrc, fdst, isrc, idst

        def start_chunk(c, par, w=W, npts=CHUNK):
            fsrc, fdst, isrc, idst = chunk_refs(c, par, w, npts)
            pltpu.async_copy(fsrc, fdst, fsems[par])
            pltpu.async_copy(isrc, idst, isems[par])

        def process_chunk(c, par, w=W, npts=CHUNK):
            fsrc, fdst, isrc, idst = chunk_refs(c, par, w, npts)
            pltpu.make_async_copy(fsrc, fdst, fsems[par]).wait()
            pltpu.make_async_copy(isrc, idst, isems[par]).wait()
            fbuf, idbuf = fbufs[par], idbufs[par]
            pt0 = pl.multiple_of(pbase + c * CHUNK, 16)
            pt0a = pl.multiple_of((pt0 // 128) * 128, 128)
            shift = pt0 - pt0a

            def group_body(k, _):
                i0 = pl.multiple_of(k * LANES, LANES)
                ids16 = idbuf[pl.ds(i0, LANES)]
                col0 = pl.multiple_of(shift + i0, 16)

                @pl.when(do_cnt)
                def _():
                    plsc.addupdate_scatter(cnt, [ids16], ones16)

                plsc.store_scatter(marks, [ids16], iota16)
                got = plsc.load_gather(marks, [ids16])
                nuniq = plsc.all_reduce_population_count(got == iota16)[0]

                for d in range(LANES):
                    f16 = fbuf[d, pl.ds(col0, LANES)]
                    a = plsc.load_gather(accs[d], [ids16])
                    plsc.store_scatter(accs[d], [ids16],
                                       jnp.maximum(a, f16))

                @pl.when(nuniq < LANES)
                def _():
                    skeys, sperm = plsc.sort_key_val(ids16, iota16)
                    tmpk[...] = skeys
                    eqm = []
                    for s, idx in zip((1, 2, 4, 8), sh_idx):
                        shk = plsc.load_gather(tmpk, [idx])
                        eqm.append((skeys == shk) & (iota16 >= s))
                    nxtk = plsc.load_gather(tmpk, [nxt_idx])
                    last = (skeys != nxtk) | (iota16 == LANES - 1)
                    for d in range(LANES):
                        f16 = fbuf[d, pl.ds(col0, LANES)]
                        tmpf[...] = f16
                        pf = plsc.load_gather(tmpf, [sperm])
                        for eq, idx in zip(eqm, sh_idx):
                            tmpf[...] = pf
                            shv = plsc.load_gather(tmpf, [idx])
                            pf = jnp.where(eq, jnp.maximum(pf, shv), pf)
                        a = plsc.load_gather(accs[d], [skeys])
                        plsc.store_scatter(accs[d], [skeys],
                                           jnp.maximum(a, pf), mask=last)
                return 0

            lax.fori_loop(0, npts // LANES, group_body, 0, unroll=2)

        start_chunk(0, 0)
        start_chunk(1, 1)

        def pair_body(g, _):
            for par in range(2):
                c = g * 2 + par
                process_chunk(c, par)

                @pl.when(c + 2 < nfull)
                def _():
                    start_chunk(c + 2, par)
                if tail:
                    @pl.when(c + 2 == nfull)
                    def _():
                        start_chunk(nfull, par, w=WT, npts=tail)
            return 0

        lax.fori_loop(0, nfull // 2, pair_body, 0)
        if nfull % 2:
            process_chunk(nfull - 1, (nfull - 1) % 2)
        if tail:
            process_chunk(nfull, nfull % 2, w=WT, npts=tail)

        obase = ((b * DG + dg) * PG + pg) * LANES * NUM_VOX
        for d in range(LANES):
            o_off = pl.multiple_of(obase + d * NUM_VOX, 128)
            pltpu.sync_copy(accs[d], outmax_ref.at[pl.ds(o_off, NUM_VOX)])

        @pl.when(do_cnt)
        def _():
            c_off = pl.multiple_of((b * PG + pg) * NUM_VOX, 128)
            pltpu.sync_copy(cnt, outcnt_ref.at[pl.ds(c_off, NUM_VOX)])

    return sc_kernel(feat, ids_flat)


def _tc_merge(partmax, partcnt, B, DG):
    def body(pm_ref, pc_ref, vf_ref, cnt_ref):
        m = jnp.maximum(pm_ref[0, 0, 0], pm_ref[0, 0, 1])
        c = pc_ref[0, 0] + pc_ref[0, 1]
        vf = jnp.where((c > 0)[None, :], m, 0.0)
        vf_ref[0] = vf.reshape(LANES, GRID3, GRID3, GRID3)
        cnt_ref[0, 0] = c

    return pl.pallas_call(
        body,
        grid=(B, DG),
        in_specs=[
            pl.BlockSpec((1, 1, PG, LANES, NUM_VOX),
                         lambda b, d: (b, d, 0, 0, 0)),
            pl.BlockSpec((1, PG, NUM_VOX), lambda b, d: (b, 0, 0)),
        ],
        out_specs=[
            pl.BlockSpec((1, LANES, GRID3, GRID3, GRID3),
                         lambda b, d: (b, d, 0, 0, 0)),
            pl.BlockSpec((1, 1, NUM_VOX), lambda b, d: (b, 0, 0)),
        ],
        out_shape=[
            jax.ShapeDtypeStruct((B, DG * LANES, GRID3, GRID3, GRID3),
                                 jnp.float32),
            jax.ShapeDtypeStruct((B, 1, NUM_VOX), jnp.int32),
        ],
    )(partmax, partcnt)


def kernel(features, voxel_ids):
    B, D, N = features.shape
    DG = D // LANES

    partmax_flat, partcnt_flat = _sc_scatter(
        features, voxel_ids.reshape(-1), B, DG, N)
    partmax = partmax_flat.reshape(B, DG, PG, LANES, NUM_VOX)
    partcnt = partcnt_flat.reshape(B, PG, NUM_VOX)
    vf3d, counts = _tc_merge(partmax, partcnt, B, DG)
    return vf3d, voxel_ids, counts
